# baseline (speedup 1.0000x reference)
import jax
import jax.numpy as jnp
from jax import lax
from jax.experimental import pallas as pl
from jax.experimental.pallas import tpu as pltpu
from jax.experimental.pallas import tpu_sc as plsc

B = 16384
D = 64
NNEG = 20
NW = 32
ROWS_PER_W = B // NW
CB = 32
NCH = ROWS_PER_W // CB
NEG_STREAMS = CB * NNEG // 128
OUTW = 32


def _halve(raw, idx2, par, nvec):
    def body(i, c):
        v = raw[pl.ds(i * 16, 16)]
        idx2[pl.ds(i * 16, 16)] = v >> 1
        par[pl.ds(i * 16, 16)] = (v & 1) * D
        return c
    lax.fori_loop(0, nvec, body, 0)


def _sc_body(pos_t, pos_c, neg2d, temb2, cemb2, dots,
             tidx, tpar, cidx, cpar, nidx, npar, tgtv, ctxv, negv, outv, sem):
    wid = lax.axis_index("s") * 2 + lax.axis_index("c")
    base = wid * ROWS_PER_W
    lane = lax.iota(jnp.int32, 16)

    pltpu.sync_copy(pos_t.at[pl.ds(base, ROWS_PER_W)], tidx)
    pltpu.sync_copy(pos_c.at[pl.ds(base, ROWS_PER_W)], cidx)
    nrows = ROWS_PER_W * NNEG // 128
    pltpu.sync_copy(neg2d.at[pl.ds(wid * nrows, nrows)], nidx)
    _halve(tidx, tidx, tpar, ROWS_PER_W // 16)
    _halve(cidx, cidx, cpar, ROWS_PER_W // 16)

    def nhalve(j, c):
        def body(i, c2):
            v = nidx[j, pl.ds(i * 16, 16)]
            nidx[j, pl.ds(i * 16, 16)] = v >> 1
            npar[pl.ds(j * 128 + i * 16, 16)] = (v & 1) * D
            return c2
        lax.fori_loop(0, 8, body, 0)
        return c
    lax.fori_loop(0, nrows, nhalve, 0)

    def chunk_body(ch, carry):
        r0 = base + ch * CB
        cps = [pltpu.async_copy(temb2.at[tidx.at[pl.ds(ch * CB, CB)]],
                                tgtv, sem),
               pltpu.async_copy(cemb2.at[cidx.at[pl.ds(ch * CB, CB)]],
                                ctxv, sem)]
        for j in range(NEG_STREAMS):
            cps.append(pltpu.async_copy(
                cemb2.at[nidx.at[ch * NEG_STREAMS + j]],
                negv.at[pl.ds(j * 128, 128)], sem))
        for cp in cps:
            cp.wait()

        def row_body(r, c2):
            rw = ch * CB + r
            ht = tpar[pl.ds(rw, 16)][0]
            hc = cpar[pl.ds(rw, 16)][0]
            t0 = tgtv[r, pl.ds(ht, 16)]
            t1 = tgtv[r, pl.ds(ht + 16, 16)]
            t2 = tgtv[r, pl.ds(ht + 32, 16)]
            t3 = tgtv[r, pl.ds(ht + 48, 16)]
            p = (t0 * ctxv[r, pl.ds(hc, 16)] + t1 * ctxv[r, pl.ds(hc + 16, 16)]
                 + t2 * ctxv[r, pl.ds(hc + 32, 16)]
                 + t3 * ctxv[r, pl.ds(hc + 48, 16)])
            rv0 = jnp.where(lane == 0, jnp.sum(p), 0.0)
            rv1 = jnp.zeros((16,), jnp.float32)
            rn = r * NNEG
            for n in range(NNEG):
                hn = npar[pl.ds(ch * CB * NNEG + rn + n, 16)][0]
                v = (t0 * negv[rn + n, pl.ds(hn, 16)]
                     + t1 * negv[rn + n, pl.ds(hn + 16, 16)]
                     + t2 * negv[rn + n, pl.ds(hn + 32, 16)]
                     + t3 * negv[rn + n, pl.ds(hn + 48, 16)])
                s = jnp.sum(v)
                if n < 15:
                    rv0 = jnp.where(lane == 1 + n, s, rv0)
                else:
                    rv1 = jnp.where(lane == n - 15, s, rv1)
            outv[r, pl.ds(0, 16)] = rv0
            outv[r, pl.ds(16, 16)] = rv1
            return c2
        lax.fori_loop(0, CB, row_body, 0)
        pltpu.sync_copy(outv, dots.at[pl.ds(r0, CB)])
        return carry
    lax.fori_loop(0, NCH, chunk_body, 0)


_sc_dots = pl.kernel(
    _sc_body,
    out_type=jax.ShapeDtypeStruct((B, OUTW), jnp.float32),
    mesh=plsc.VectorSubcoreMesh(core_axis_name="c", subcore_axis_name="s"),
    compiler_params=pltpu.CompilerParams(needs_layout_passes=False),
    scratch_types=[
        pltpu.VMEM((ROWS_PER_W,), jnp.int32),
        pltpu.VMEM((ROWS_PER_W + 16,), jnp.int32),
        pltpu.VMEM((ROWS_PER_W,), jnp.int32),
        pltpu.VMEM((ROWS_PER_W + 16,), jnp.int32),
        pltpu.VMEM((ROWS_PER_W * NNEG // 128, 128), jnp.int32),
        pltpu.VMEM((ROWS_PER_W * NNEG + 16,), jnp.int32),
        pltpu.VMEM((CB, 2 * D), jnp.float32),
        pltpu.VMEM((CB, 2 * D), jnp.float32),
        pltpu.VMEM((CB * NNEG, 2 * D), jnp.float32),
        pltpu.VMEM((CB, OUTW), jnp.float32),
        pltpu.SemaphoreType.DMA,
    ],
)


def _tc_loss_body(d_ref, o_ref):
    x = d_ref[:]
    col = lax.broadcasted_iota(jnp.int32, x.shape, 1) % OUTW
    xc = jnp.clip(x, -10.0, 10.0)
    pos_f = jnp.log1p(jnp.exp(-xc))
    neg_f = jnp.log1p(jnp.exp(xc))
    contrib = jnp.where(col == 0, pos_f,
                        jnp.where(col <= NNEG, neg_f, 0.0))
    o_ref[0, 0] = jnp.sum(contrib) * (1.0 / B)


_tc_loss = pl.pallas_call(
    _tc_loss_body,
    out_shape=jax.ShapeDtypeStruct((1, 1), jnp.float32),
    in_specs=[pl.BlockSpec(memory_space=pltpu.VMEM)],
    out_specs=pl.BlockSpec(memory_space=pltpu.SMEM),
)


def kernel(pos_target, pos_context, neg_context, target_emb, context_emb):
    temb2 = target_emb.reshape(-1, 2 * D)
    cemb2 = context_emb.reshape(-1, 2 * D)
    neg2d = neg_context.reshape(B * NNEG // 128, 128)
    dots = _sc_dots(pos_target, pos_context, neg2d, temb2, cemb2)
    loss = _tc_loss(dots.reshape(B * OUTW // 128, 128))
    return loss[0, 0]


# speedup vs baseline: 1.1177x; 1.1177x over previous
import jax
import jax.numpy as jnp
from jax import lax
from jax.experimental import pallas as pl
from jax.experimental.pallas import tpu as pltpu
from jax.experimental.pallas import tpu_sc as plsc

B = 16384
D = 64
NNEG = 20
NW = 32
ROWS_PER_W = B // NW
CB = 32
NCH = ROWS_PER_W // CB
NSTR = CB * NNEG // 128
NIDX_ROWS = ROWS_PER_W * NNEG // 128
OUTW = 32


def _sc_body(pos_t, pos_c, neg2d, temb, cemb, dots,
             tidx, cidx, nidx, tgtv, ctxv, negv, outv, sem0, sem1):
    wid = lax.axis_index("s") * 2 + lax.axis_index("c")
    base = wid * ROWS_PER_W
    lane = lax.iota(jnp.int32, 16)
    sems = (sem0, sem1)

    pltpu.sync_copy(pos_t.at[pl.ds(base, ROWS_PER_W)], tidx)
    pltpu.sync_copy(pos_c.at[pl.ds(base, ROWS_PER_W)], cidx)
    pltpu.sync_copy(neg2d.at[pl.ds(wid * NIDX_ROWS, NIDX_ROWS)], nidx)

    def issue(ch, b):
        pltpu.async_copy(temb.at[tidx.at[pl.ds(ch * CB, CB)]],
                         tgtv.at[pl.ds(b * CB, CB)], sems[b])
        pltpu.async_copy(cemb.at[cidx.at[pl.ds(ch * CB, CB)]],
                         ctxv.at[pl.ds(b * CB, CB)], sems[b])
        for j in range(NSTR):
            pltpu.async_copy(cemb.at[nidx.at[ch * NSTR + j]],
                             negv.at[pl.ds((b * NSTR + j) * 128, 128)],
                             sems[b])

    def drain(b):
        pltpu.make_async_copy(temb.at[pl.ds(0, CB)],
                              tgtv.at[pl.ds(b * CB, CB)], sems[b]).wait()
        pltpu.make_async_copy(cemb.at[pl.ds(0, CB)],
                              ctxv.at[pl.ds(b * CB, CB)], sems[b]).wait()
        for j in range(NSTR):
            pltpu.make_async_copy(
                cemb.at[pl.ds(0, 128)],
                negv.at[pl.ds((b * NSTR + j) * 128, 128)], sems[b]).wait()

    def compute(ch, b):
        def row_body(r, c2):
            rb = b * CB + r
            t0 = tgtv[rb, pl.ds(0, 16)]
            t1 = tgtv[rb, pl.ds(16, 16)]
            t2 = tgtv[rb, pl.ds(32, 16)]
            t3 = tgtv[rb, pl.ds(48, 16)]
            p = (t0 * ctxv[rb, pl.ds(0, 16)] + t1 * ctxv[rb, pl.ds(16, 16)]
                 + t2 * ctxv[rb, pl.ds(32, 16)]
                 + t3 * ctxv[rb, pl.ds(48, 16)])
            rv0 = jnp.where(lane == 0, jnp.sum(p), 0.0)
            rv1 = jnp.zeros((16,), jnp.float32)
            rn = b * CB * NNEG + r * NNEG
            for n in range(NNEG):
                v = (t0 * negv[rn + n, pl.ds(0, 16)]
                     + t1 * negv[rn + n, pl.ds(16, 16)]
                     + t2 * negv[rn + n, pl.ds(32, 16)]
                     + t3 * negv[rn + n, pl.ds(48, 16)])
                s = jnp.sum(v)
                if n < 15:
                    rv0 = jnp.where(lane == 1 + n, s, rv0)
                else:
                    rv1 = jnp.where(lane == n - 15, s, rv1)
            outv[r, pl.ds(0, 16)] = rv0
            outv[r, pl.ds(16, 16)] = rv1
            return c2
        lax.fori_loop(0, CB, row_body, 0)
        pltpu.sync_copy(outv, dots.at[pl.ds(base + ch * CB, CB)])

    issue(0, 0)
    issue(1, 1)

    def pair_body(i, carry):
        ch0 = i * 2
        for b in range(2):
            drain(b)
            compute(ch0 + b, b)
            issue(ch0 + b + 2, b)
        return carry
    lax.fori_loop(0, NCH // 2 - 1, pair_body, 0)
    for b in range(2):
        drain(b)
        compute(NCH - 2 + b, b)


_sc_dots = pl.kernel(
    _sc_body,
    out_type=jax.ShapeDtypeStruct((B, OUTW), jnp.float32),
    mesh=plsc.VectorSubcoreMesh(core_axis_name="c", subcore_axis_name="s"),
    compiler_params=pltpu.CompilerParams(needs_layout_passes=False,
                                         use_tc_tiling_on_sc=False),
    scratch_types=[
        pltpu.VMEM((ROWS_PER_W,), jnp.int32),
        pltpu.VMEM((ROWS_PER_W,), jnp.int32),
        pltpu.VMEM((NIDX_ROWS, 128), jnp.int32),
        pltpu.VMEM((2 * CB, D), jnp.float32),
        pltpu.VMEM((2 * CB, D), jnp.float32),
        pltpu.VMEM((2 * CB * NNEG, D), jnp.float32),
        pltpu.VMEM((CB, OUTW), jnp.float32),
        pltpu.SemaphoreType.DMA,
        pltpu.SemaphoreType.DMA,
    ],
)


def _tc_loss_body(d_ref, o_ref):
    x = d_ref[:]
    col = lax.broadcasted_iota(jnp.int32, x.shape, 1) % OUTW
    xc = jnp.clip(x, -10.0, 10.0)
    pos_f = jnp.log1p(jnp.exp(-xc))
    neg_f = jnp.log1p(jnp.exp(xc))
    contrib = jnp.where(col == 0, pos_f,
                        jnp.where(col <= NNEG, neg_f, 0.0))
    o_ref[0, 0] = jnp.sum(contrib) * (1.0 / B)


_tc_loss = pl.pallas_call(
    _tc_loss_body,
    out_shape=jax.ShapeDtypeStruct((1, 1), jnp.float32),
    in_specs=[pl.BlockSpec(memory_space=pltpu.VMEM)],
    out_specs=pl.BlockSpec(memory_space=pltpu.SMEM),
)


def kernel(pos_target, pos_context, neg_context, target_emb, context_emb):
    neg2d = neg_context.reshape(B * NNEG // 128, 128)
    dots = _sc_dots(pos_target, pos_context, neg2d, target_emb, context_emb)
    loss = _tc_loss(dots.reshape(B * OUTW // 128, 128))
    return loss[0, 0]
